# Initial kernel scaffold; baseline (speedup 1.0000x reference)
import jax
import jax.numpy as jnp
from jax.experimental import pallas as pl


def kernel(reference_features, query_features):
    raise NotImplementedError("write your pallas kernel here")


# speedup vs baseline: 1.2751x; 1.2751x over previous
import functools

import jax
import jax.numpy as jnp
from jax.experimental import pallas as pl
from jax.experimental.pallas import tpu as pltpu

_MAX_DISP = 4
_NS = 2 * _MAX_DISP + 1


def _corr_kernel(ref_ref, q_ref, out_ref, *, inv_c):
    k = pl.program_id(1)
    _, cc, h, w = ref_ref.shape

    @pl.when(k == 0)
    def _():
        out_ref[...] = jnp.zeros_like(out_ref)

    r = ref_ref[0] * inv_c
    for dy in range(_NS):
        for dx in range(_NS):
            i = dy * _NS + dx
            qs = q_ref[0, :, dy:dy + h, dx:dx + w]
            out_ref[0, i] += jnp.sum(r * qs, axis=0)


def kernel(reference_features, query_features):
    b, c, h, w = reference_features.shape
    p = _MAX_DISP
    q = jnp.pad(query_features, ((0, 0), (0, 0), (p, p), (p, p)))

    cc = min(32, c)
    n_chunks = c // cc
    d = _NS * _NS

    return pl.pallas_call(
        functools.partial(_corr_kernel, inv_c=1.0 / c),
        grid=(b, n_chunks),
        in_specs=[
            pl.BlockSpec((1, cc, h, w), lambda bi, ki: (bi, ki, 0, 0)),
            pl.BlockSpec((1, cc, h + 2 * p, w + 2 * p),
                         lambda bi, ki: (bi, ki, 0, 0)),
        ],
        out_specs=pl.BlockSpec((1, d, h, w), lambda bi, ki: (bi, 0, 0, 0)),
        out_shape=jax.ShapeDtypeStruct((b, d, h, w), jnp.float32),
        compiler_params=pltpu.CompilerParams(
            dimension_semantics=("parallel", "arbitrary"),
            vmem_limit_bytes=56 * 1024 * 1024,
        ),
        name="corr_cost_volume",
    )(reference_features, q)


# speedup vs baseline: 2.7253x; 2.1373x over previous
import functools

import jax
import jax.numpy as jnp
from jax.experimental import pallas as pl
from jax.experimental.pallas import tpu as pltpu

_MAX_DISP = 4
_NS = 2 * _MAX_DISP + 1


def _corr_kernel(ref_ref, q_ref, out_ref, qx_ref, *, n_chunks, inv_c):
    k = pl.program_id(1)
    _, cc, h, w = ref_ref.shape

    @pl.when(k == 0)
    def _():
        out_ref[...] = jnp.zeros_like(out_ref)

    for dx in range(_NS):
        slot = dx % 2
        qx_ref[slot] = q_ref[0, :, :, dx:dx + w]
        for dy in range(_NS):
            i = dy * _NS + dx
            prod = ref_ref[0] * qx_ref[slot, :, dy:dy + h, :]
            out_ref[0, i] += jnp.sum(prod, axis=0)

    @pl.when(k == n_chunks - 1)
    def _():
        out_ref[...] = out_ref[...] * inv_c


def kernel(reference_features, query_features):
    b, c, h, w = reference_features.shape
    p = _MAX_DISP
    q = jnp.pad(query_features, ((0, 0), (0, 0), (p, p), (p, p)))

    cc = min(32, c)
    n_chunks = c // cc
    d = _NS * _NS

    return pl.pallas_call(
        functools.partial(_corr_kernel, n_chunks=n_chunks, inv_c=1.0 / c),
        grid=(b, n_chunks),
        in_specs=[
            pl.BlockSpec((1, cc, h, w), lambda bi, ki: (bi, ki, 0, 0)),
            pl.BlockSpec((1, cc, h + 2 * p, w + 2 * p),
                         lambda bi, ki: (bi, ki, 0, 0)),
        ],
        out_specs=pl.BlockSpec((1, d, h, w), lambda bi, ki: (bi, 0, 0, 0)),
        out_shape=jax.ShapeDtypeStruct((b, d, h, w), jnp.float32),
        scratch_shapes=[
            pltpu.VMEM((2, cc, h + 2 * p, w), jnp.float32),
        ],
        compiler_params=pltpu.CompilerParams(
            dimension_semantics=("parallel", "arbitrary"),
            vmem_limit_bytes=56 * 1024 * 1024,
        ),
        name="corr_cost_volume",
    )(reference_features, q)
